# baseline (speedup 1.0000x reference)
import functools

import jax
import jax.numpy as jnp
from jax import lax
from jax.experimental import pallas as pl
from jax.experimental.pallas import tpu as pltpu
from jax.experimental.pallas import tpu_sc as plsc

_TOPK = 5
_B = 4
_V = 32
_ROW = 128 * 128 * 3
_NC = 2
_NS = 16
_NEG = -1e30
_BIG = 1 << 30


def _body(sel_hbm, img_hbm, pose_hbm, out_img, out_pose, out_prob,
          sel_v, img_v, pose_v, prob_v):
    wid = lax.axis_index("s") * _NC + lax.axis_index("c")

    @pl.when(wid < _B * _TOPK)
    def _():
        pltpu.sync_copy(sel_hbm, sel_v)
        b = wid // _TOPK
        k = wid % _TOPK
        iota = lax.iota(jnp.int32, 16)
        w0 = sel_v[pl.ds(b * _V, 16)]
        w1 = sel_v[pl.ds(b * _V + 16, 16)]

        vals = jnp.full((16,), _NEG, jnp.float32)
        idx_own = 0
        m_glob = None
        for t in range(_TOPK):
            m0 = jnp.max(w0)
            m1 = jnp.max(w1)
            use0 = m0 >= m1
            i0 = jnp.min(jnp.where(w0 == m0, iota, _BIG))
            i1 = jnp.min(jnp.where(w1 == m1, iota, _BIG))
            idx_t = jnp.where(use0, i0, i1 + 16)
            m_t = jnp.where(use0, m0, m1)
            if t == 0:
                m_glob = m_t
            w0 = jnp.where((iota == i0) & use0, _NEG, w0)
            w1 = jnp.where((iota == i1) & jnp.logical_not(use0), _NEG, w1)
            vals = jnp.where(iota == t, m_t, vals)
            idx_own = jnp.where(k == t, idx_t, idx_own)

        e = jnp.exp(vals - m_glob)
        e = jnp.where(iota < _TOPK, e, 0.0)
        probs = e / jnp.sum(e)

        g = b * _V + idx_own
        pltpu.sync_copy(img_hbm.at[g], img_v)
        pltpu.sync_copy(img_v, out_img.at[wid])
        pltpu.sync_copy(pose_hbm.at[g], pose_v)
        pltpu.sync_copy(pose_v, out_pose.at[wid])

        @pl.when(k == 0)
        def _():
            prob_v[pl.ds(0, 16)] = probs
            pltpu.sync_copy(prob_v, out_prob.at[b])


_sc_call = pl.kernel(
    _body,
    out_type=(
        jax.ShapeDtypeStruct((_B * _TOPK, _ROW), jnp.float32),
        jax.ShapeDtypeStruct((_B * _TOPK, 128), jnp.float32),
        jax.ShapeDtypeStruct((_B, 128), jnp.float32),
    ),
    mesh=plsc.VectorSubcoreMesh(core_axis_name="c", subcore_axis_name="s"),
    scratch_types=[
        pltpu.VMEM((_B * _V,), jnp.float32),
        pltpu.VMEM((_ROW,), jnp.float32),
        pltpu.VMEM((128,), jnp.float32),
        pltpu.VMEM((128,), jnp.float32),
    ],
    compiler_params=pltpu.CompilerParams(needs_layout_passes=False),
)


@jax.jit
def kernel(selection_weights, images, poses):
    sel = selection_weights.reshape(_B * _V)
    img = images.reshape(_B * _V, _ROW)
    pose = jnp.pad(poses, ((0, 0), (0, 0), (0, 121))).reshape(_B * _V, 128)
    out_img, out_pose, out_prob = _sc_call(sel, img, pose)
    return (
        out_img.reshape(_B, _TOPK, 128, 128, 3),
        out_pose[:, :7].reshape(_B, _TOPK, 7),
        out_prob[:, :_TOPK],
    )
